# baseline (speedup 1.0000x reference)
import functools

import jax
import jax.numpy as jnp
from jax import lax
from jax.experimental import pallas as pl
from jax.experimental.pallas import tpu as pltpu
from jax.experimental.pallas import tpu_sc as plsc

N = 10000
C = 64
E = 320000
NIDX = 5000
ALPHA = 0.1
NITER = 10

NCORES = 2
NSUB = 16
NW = NCORES * NSUB
NP = 10240
RPT = NP // NSUB
BLK = 128
GRP = 2
YROWS = 10016
STEPS = 80
EP = NW * STEPS * BLK

NIDXP = 5120
IPW = NIDXP // NW
IBLK = 80

HALF = NP // 2
CAPB = 52
CAP = CAPB * BLK

_sc_mesh = plsc.VectorSubcoreMesh(core_axis_name="c", subcore_axis_name="s")
_sc_params = pltpu.CompilerParams(needs_layout_passes=False,
                                  use_tc_tiling_on_sc=False)


def _deg_body(src_hbm, dst_hbm, out_hbm, srcp_hbm, dstp_hbm, nblk_hbm,
              src_v, dst_v, hist, ls, ld, hs, hd, cbuf):
    cid = lax.axis_index("c")
    sid = lax.axis_index("s")
    wid = cid * NSUB + sid

    zv = jnp.zeros((16,), jnp.float32)

    def zh(i, _):
        hist[pl.ds(i * 16, 16)] = zv
        return 0

    lax.fori_loop(0, NP // 16, zh, 0)
    pltpu.sync_copy(src_hbm.at[wid], src_v)
    pltpu.sync_copy(dst_hbm.at[wid], dst_v)

    ones = jnp.ones((16,), jnp.float32)
    sub = BLK // 16
    nmax = CAP - 1040

    def st(i, lohi):
        lo, hi = lohi
        j = i // sub
        k = lax.rem(i, sub)
        sl = pl.ds(k * 16, 16)
        dv = dst_v[j, sl]
        sv = src_v[j, sl]
        plsc.addupdate_scatter(hist, [dv], ones)
        mlo = dv < HALF
        mhi = jnp.logical_not(mlo)
        plsc.store_compressed(ls.at[pl.ds(lo, 16)], sv, mask=mlo)
        plsc.store_compressed(ld.at[pl.ds(lo, 16)], dv, mask=mlo)
        plsc.store_compressed(hs.at[pl.ds(hi, 16)], sv, mask=mhi)
        plsc.store_compressed(hd.at[pl.ds(hi, 16)], dv - HALF, mask=mhi)
        nlo = jnp.sum(mlo.astype(jnp.int32))
        lo = jnp.minimum(lo + nlo, nmax)
        hi = jnp.minimum(hi + (16 - nlo), nmax)
        return lo, hi

    lo, hi = lax.fori_loop(0, STEPS * sub, st, (0, 0))

    padsrc = jnp.full((16,), N, jnp.int32)
    paddst = jnp.zeros((16,), jnp.int32)
    for k in range(64):
        ls[pl.ds(lo + k * 16, 16)] = padsrc
        ld[pl.ds(lo + k * 16, 16)] = paddst
        hs[pl.ds(hi + k * 16, 16)] = padsrc
        hd[pl.ds(hi + k * 16, 16)] = paddst
    nblo = ((lo + BLK - 1) // BLK + 7) & ~7
    nbhi = ((hi + BLK - 1) // BLK + 7) & ~7

    pltpu.sync_copy(hist, out_hbm.at[wid])
    pltpu.sync_copy(ls, srcp_hbm.at[0, wid])
    pltpu.sync_copy(ld, dstp_hbm.at[0, wid])
    pltpu.sync_copy(hs, srcp_hbm.at[1, wid])
    pltpu.sync_copy(hd, dstp_hbm.at[1, wid])
    cbuf[...] = jnp.broadcast_to(nblo, (16,))
    pltpu.sync_copy(cbuf, nblk_hbm.at[0, wid])
    cbuf[...] = jnp.broadcast_to(nbhi, (16,))
    pltpu.sync_copy(cbuf, nblk_hbm.at[1, wid])


_deg_call = functools.partial(
    pl.kernel,
    _deg_body,
    out_type=(jax.ShapeDtypeStruct((NW, NP), jnp.float32),
              jax.ShapeDtypeStruct((2, NW, CAP), jnp.int32),
              jax.ShapeDtypeStruct((2, NW, CAP), jnp.int32),
              jax.ShapeDtypeStruct((2, NW, 16), jnp.int32)),
    mesh=_sc_mesh,
    scratch_types=[
        pltpu.VMEM((STEPS, BLK), jnp.int32),
        pltpu.VMEM((STEPS, BLK), jnp.int32),
        pltpu.VMEM((NP,), jnp.float32),
        pltpu.VMEM((CAP,), jnp.int32),
        pltpu.VMEM((CAP,), jnp.int32),
        pltpu.VMEM((CAP,), jnp.int32),
        pltpu.VMEM((CAP,), jnp.int32),
        pltpu.VMEM((16,), jnp.int32),
    ],
    compiler_params=_sc_params,
)()


CH = 16
OHR0 = (YROWS - HALF) // NSUB
OHR1 = HALF // NSUB


def _prop_body(y0_hbm, a1x_hbm, bv_hbm, srcp_hbm, dstp_hbm, nblk_hbm,
               yout_hbm, srca, dsta, srcb, dstb, rows_v, zbuf,
               abuf, ybuf, a1buf, bvbuf, nbuf,
               y_sh, acc, sga, sgb, ssa, ssb, xsem):
    cid = lax.axis_index("c")
    sid = lax.axis_index("s")
    arpt = HALF // NSUB
    yrpt = YROWS // NSUB

    pltpu.sync_copy(y0_hbm.at[pl.ds(sid * yrpt, yrpt), :],
                    y_sh.at[pl.ds(sid * yrpt, yrpt), :])

    zv = jnp.zeros((16,), jnp.float32)
    csub = C // 16

    def zz(i, _):
        r = i // csub
        cc = lax.rem(i, csub)
        zbuf[r, pl.ds(cc * 16, 16)] = zv
        return 0

    lax.fori_loop(0, CH * csub, zz, 0)

    def zrow(i, _):
        pltpu.sync_copy(zbuf, acc.at[pl.ds(sid * arpt + i * CH, CH), :])
        return 0

    lax.fori_loop(0, arpt // CH, zrow, 0)

    pltpu.sync_copy(srcp_hbm.at[cid, 2 * sid], srca)
    pltpu.sync_copy(dstp_hbm.at[cid, 2 * sid], dsta)
    pltpu.sync_copy(srcp_hbm.at[cid, 2 * sid + 1], srcb)
    pltpu.sync_copy(dstp_hbm.at[cid, 2 * sid + 1], dstb)
    pltpu.sync_copy(nblk_hbm.at[cid, 2 * sid], nbuf)
    na = nbuf[pl.ds(0, 16)][0]
    pltpu.sync_copy(nblk_hbm.at[cid, 2 * sid + 1], nbuf)
    nb = nbuf[pl.ds(0, 16)][0]
    plsc.subcore_barrier()

    def rslice(half, k):
        return rows_v.at[pl.ds((half * GRP + k) * BLK, BLK), :]

    def run_list(src_l, dst_l, nblk):
        ngrp = nblk // GRP

        def fire_gather(half, g, sm):
            for k in range(GRP):
                pltpu.async_copy(y_sh.at[src_l.at[g * GRP + k]],
                                 rslice(half, k), sm)

        def drain_gather(half, sm):
            for k in range(GRP):
                pltpu.make_async_copy(y0_hbm.at[src_l.at[0]],
                                      rslice(half, k), sm).wait()

        def fire_scatter(half, g, sm):
            for k in range(GRP):
                pltpu.async_copy(rslice(half, k),
                                 acc.at[dst_l.at[g * GRP + k]], sm, add=True)

        def drain_scatter(half, sm):
            for k in range(GRP):
                pltpu.make_async_copy(rslice(half, k),
                                      acc.at[dst_l.at[0]], sm).wait()

        @pl.when(ngrp > 0)
        def _():
            fire_gather(0, 0, sga)

            def pair(i, _):
                fire_gather(1, 2 * i + 1, sgb)
                drain_gather(0, sga)
                fire_scatter(0, 2 * i, ssa)
                drain_gather(1, sgb)
                fire_scatter(1, 2 * i + 1, ssb)
                drain_scatter(0, ssa)

                @pl.when(2 * i + 2 < ngrp)
                def _():
                    fire_gather(0, 2 * i + 2, sga)

                drain_scatter(1, ssb)
                return 0

            lax.fori_loop(0, ngrp // 2, pair, 0)

    def xbarrier():
        @pl.when(sid == 0)
        def _():
            pltpu.semaphore_signal(xsem, 1, core_index=1 - cid)
            pltpu.semaphore_wait(xsem, 1)

        plsc.subcore_barrier()

    def iteration(it, _):
        run_list(srca, dsta, na)
        run_list(srcb, dstb, nb)
        plsc.subcore_barrier()

        def comb(ch, _):
            rel = sid * arpt + ch * CH
            glob = cid * HALF + rel

            @pl.when(glob + CH <= YROWS)
            def _():
                pltpu.sync_copy(acc.at[pl.ds(rel, CH), :], abuf)
                pltpu.sync_copy(y_sh.at[pl.ds(glob, CH), :], ybuf)
                pltpu.sync_copy(a1x_hbm.at[pl.ds(glob, CH), :], a1buf)
                pltpu.sync_copy(bv_hbm.at[pl.ds(glob, CH), :], bvbuf)
                for rr in range(CH):
                    for cc in range(csub):
                        sl = pl.ds(cc * 16, 16)
                        s = abuf[rr, sl] + ybuf[rr, sl]
                        ybuf[rr, sl] = a1buf[rr, sl] * s + bvbuf[rr, sl]
                pltpu.sync_copy(ybuf, y_sh.at[pl.ds(glob, CH), :])
                pltpu.sync_copy(ybuf, yout_hbm.at[pl.ds(glob, CH), :])

            pltpu.sync_copy(zbuf, acc.at[pl.ds(rel, CH), :])
            return 0

        lax.fori_loop(0, arpt // CH, comb, 0)
        plsc.subcore_barrier()
        xbarrier()

        @pl.when(cid == 0)
        def _():
            pltpu.sync_copy(
                yout_hbm.at[pl.ds(HALF + sid * OHR0, OHR0), :],
                y_sh.at[pl.ds(HALF + sid * OHR0, OHR0), :])

        @pl.when(cid == 1)
        def _():
            pltpu.sync_copy(yout_hbm.at[pl.ds(sid * OHR1, OHR1), :],
                            y_sh.at[pl.ds(sid * OHR1, OHR1), :])

        plsc.subcore_barrier()
        xbarrier()
        return 0

    lax.fori_loop(0, NITER, iteration, 0)


_prop_call = functools.partial(
    pl.kernel,
    _prop_body,
    out_type=jax.ShapeDtypeStruct((NP, C), jnp.float32),
    mesh=_sc_mesh,
    scratch_types=[
        pltpu.VMEM((CAPB, BLK), jnp.int32),
        pltpu.VMEM((CAPB, BLK), jnp.int32),
        pltpu.VMEM((CAPB, BLK), jnp.int32),
        pltpu.VMEM((CAPB, BLK), jnp.int32),
        pltpu.VMEM((2 * GRP * BLK, C), jnp.float32),
        pltpu.VMEM((CH, C), jnp.float32),
        pltpu.VMEM((CH, C), jnp.float32),
        pltpu.VMEM((CH, C), jnp.float32),
        pltpu.VMEM((CH, C), jnp.float32),
        pltpu.VMEM((CH, C), jnp.float32),
        pltpu.VMEM((16,), jnp.int32),
        pltpu.VMEM_SHARED((YROWS, C), jnp.float32),
        pltpu.VMEM_SHARED((HALF, C), jnp.float32),
        pltpu.SemaphoreType.DMA,
        pltpu.SemaphoreType.DMA,
        pltpu.SemaphoreType.DMA,
        pltpu.SemaphoreType.DMA,
        pltpu.SemaphoreType.REGULAR,
    ],
    compiler_params=_sc_params,
)()


def _gather_body(y_hbm, idx_hbm, iv_hbm, yg_hbm, ivg_hbm, idx_v, rows_v,
                 iv_v, ob_v):
    cid = lax.axis_index("c")
    sid = lax.axis_index("s")
    wid = cid * NSUB + sid

    pltpu.sync_copy(idx_hbm.at[wid], idx_v)
    pltpu.sync_copy(iv_hbm, iv_v)
    for t in range(IPW // IBLK):
        pltpu.sync_copy(y_hbm.at[idx_v.at[t]], rows_v)
        pltpu.sync_copy(rows_v,
                        yg_hbm.at[pl.ds(wid * IPW + t * IBLK, IBLK), :])
        for k in range(IBLK // 16):
            v = idx_v[t, pl.ds(k * 16, 16)]
            g = plsc.load_gather(iv_v, [v])
            ob_v[pl.ds((t * (IBLK // 16) + k) * 16, 16)] = g
    pltpu.sync_copy(ob_v, ivg_hbm.at[pl.ds(wid * IPW, IPW)])


_gather_call = functools.partial(
    pl.kernel,
    _gather_body,
    out_type=(jax.ShapeDtypeStruct((NIDXP, C), jnp.float32),
              jax.ShapeDtypeStruct((NIDXP,), jnp.float32)),
    mesh=_sc_mesh,
    scratch_types=[
        pltpu.VMEM((IPW // IBLK, IBLK), jnp.int32),
        pltpu.VMEM((IBLK, C), jnp.float32),
        pltpu.VMEM((NP,), jnp.float32),
        pltpu.VMEM((IPW,), jnp.float32),
    ],
    compiler_params=_sc_params,
)()


_MB = 512


def _mlp_body(x_ref, w1_ref, w2_ref, w3_ref, o_ref):
    h = jnp.maximum(
        jnp.dot(x_ref[...], w1_ref[...], preferred_element_type=jnp.float32),
        0.0)
    h = jnp.maximum(
        jnp.dot(h, w2_ref[...], preferred_element_type=jnp.float32), 0.0)
    o_ref[...] = jnp.dot(h, w3_ref[...], preferred_element_type=jnp.float32)


def _mlp_call(x, w1, w2, w3):
    f = x.shape[1]
    h1, h2 = w2.shape[0], w3.shape[0]
    return pl.pallas_call(
        _mlp_body,
        grid=(NP // _MB,),
        in_specs=[
            pl.BlockSpec((_MB, f), lambda i: (i, 0)),
            pl.BlockSpec((f, h1), lambda i: (0, 0)),
            pl.BlockSpec((h1, h2), lambda i: (0, 0)),
            pl.BlockSpec((h2, C), lambda i: (0, 0)),
        ],
        out_specs=pl.BlockSpec((_MB, C), lambda i: (i, 0)),
        out_shape=jax.ShapeDtypeStruct((NP, C), jnp.float32),
    )(x, w1, w2, w3)


_RB = 512


def _prep_body(hist_ref, l_ref, a1_ref, bv_ref, y0_ref, iv_ref):
    i = pl.program_id(0)
    cnt = jnp.sum(hist_ref[...], axis=1, keepdims=True)
    deg = cnt + 1.0
    dis = lax.rsqrt(deg)
    row = i * _RB + lax.broadcasted_iota(jnp.int32, (_RB, 1), 0)
    mask = (row < N).astype(jnp.float32)
    lv = l_ref[...]
    a1_ref[...] = (1.0 - ALPHA) * dis * dis * mask
    bv_ref[...] = ALPHA * dis * lv
    y0_ref[...] = dis * lv
    iv_ref[...] = jnp.sqrt(deg)


def _prep_call(hist_t, l):
    return pl.pallas_call(
        _prep_body,
        grid=(NP // _RB,),
        in_specs=[
            pl.BlockSpec((_RB, NW), lambda i: (i, 0)),
            pl.BlockSpec((_RB, C), lambda i: (i, 0)),
        ],
        out_specs=[
            pl.BlockSpec((_RB, 1), lambda i: (i, 0)),
            pl.BlockSpec((_RB, C), lambda i: (i, 0)),
            pl.BlockSpec((_RB, C), lambda i: (i, 0)),
            pl.BlockSpec((_RB, 1), lambda i: (i, 0)),
        ],
        out_shape=[
            jax.ShapeDtypeStruct((NP, 1), jnp.float32),
            jax.ShapeDtypeStruct((NP, C), jnp.float32),
            jax.ShapeDtypeStruct((NP, C), jnp.float32),
            jax.ShapeDtypeStruct((NP, 1), jnp.float32),
        ],
    )(hist_t, l)


_CB = 1024


def _comb_body(p_ref, y_ref, a1_ref, bv_ref, o_ref):
    s = p_ref[...] + y_ref[...]
    o_ref[...] = a1_ref[...] * s + bv_ref[...]


def _comb_call(p, y, a1, bv):
    return pl.pallas_call(
        _comb_body,
        grid=(NP // _CB,),
        in_specs=[
            pl.BlockSpec((_CB, C), lambda i: (i, 0)),
            pl.BlockSpec((_CB, C), lambda i: (i, 0)),
            pl.BlockSpec((_CB, 1), lambda i: (i, 0)),
            pl.BlockSpec((_CB, C), lambda i: (i, 0)),
        ],
        out_specs=pl.BlockSpec((_CB, C), lambda i: (i, 0)),
        out_shape=jax.ShapeDtypeStruct((NP, C), jnp.float32),
    )(p, y, a1, bv)


_SB = 512


def _lsm_body(y_ref, iv_ref, o_ref):
    z = y_ref[...] * iv_ref[...]
    m = jnp.max(z, axis=1, keepdims=True)
    e = jnp.exp(z - m)
    s = jnp.sum(e, axis=1, keepdims=True)
    o_ref[...] = (z - m) - jnp.log(s)


def _lsm_call(yg, ivg):
    return pl.pallas_call(
        _lsm_body,
        grid=(NIDXP // _SB,),
        in_specs=[
            pl.BlockSpec((_SB, C), lambda i: (i, 0)),
            pl.BlockSpec((_SB, 1), lambda i: (i, 0)),
        ],
        out_specs=pl.BlockSpec((_SB, C), lambda i: (i, 0)),
        out_shape=jax.ShapeDtypeStruct((NIDXP, C), jnp.float32),
    )(yg, ivg)


def kernel(attr_matrix, idx, W1, W2, W3, edge_index):
    attr_p = jnp.pad(attr_matrix, ((0, NP - N), (0, 0)))
    src = jnp.concatenate(
        [edge_index[0], jnp.zeros((EP - E,), jnp.int32)]).reshape(
            NW, STEPS, BLK)
    dst = jnp.concatenate(
        [edge_index[1], jnp.full((EP - E,), N, jnp.int32)]).reshape(
            NW, STEPS, BLK)
    idx_p = jnp.concatenate(
        [idx, jnp.zeros((NIDXP - NIDX,), jnp.int32)]).reshape(
            NW, IPW // IBLK, IBLK)

    l = _mlp_call(attr_p, W1, W2, W3)
    hist, srcp, dstp, nblk = _deg_call(src, dst)
    srcp = srcp.reshape(2, NW, CAPB, BLK)
    dstp = dstp.reshape(2, NW, CAPB, BLK)
    a1, bv, y0, iv = _prep_call(hist.T, l)
    a1x = jnp.broadcast_to(a1, (NP, C))
    y = _prop_call(y0, a1x, bv, srcp, dstp, nblk)
    yg, ivg = _gather_call(y, idx_p, iv.reshape(NP))
    out = _lsm_call(yg, ivg.reshape(NIDXP, 1))
    return out[:NIDX]


# speedup vs baseline: 1.0707x; 1.0707x over previous
import functools

import jax
import jax.numpy as jnp
from jax import lax
from jax.experimental import pallas as pl
from jax.experimental.pallas import tpu as pltpu
from jax.experimental.pallas import tpu_sc as plsc

N = 10000
C = 64
E = 320000
NIDX = 5000
ALPHA = 0.1
NITER = 10

NCORES = 2
NSUB = 16
NW = NCORES * NSUB
NP = 10240
RPT = NP // NSUB
BLK = 128
GRP = 2
YROWS = 10016
STEPS = 80
EP = NW * STEPS * BLK

NIDXP = 5120
IPW = NIDXP // NW
IBLK = 80

HALF = NP // 2
CAPB = 52
CAP = CAPB * BLK

_sc_mesh = plsc.VectorSubcoreMesh(core_axis_name="c", subcore_axis_name="s")
_sc_params = pltpu.CompilerParams(needs_layout_passes=False,
                                  use_tc_tiling_on_sc=False)


def _deg_body(src_hbm, dst_hbm, out_hbm, srcp_hbm, dstp_hbm, nblk_hbm,
              src_v, dst_v, hist, ls, ld, hs, hd, cbuf):
    cid = lax.axis_index("c")
    sid = lax.axis_index("s")
    wid = cid * NSUB + sid

    zv = jnp.zeros((16,), jnp.float32)

    def zh(i, _):
        hist[pl.ds(i * 16, 16)] = zv
        return 0

    lax.fori_loop(0, NP // 16, zh, 0)
    pltpu.sync_copy(src_hbm.at[wid], src_v)
    pltpu.sync_copy(dst_hbm.at[wid], dst_v)

    ones = jnp.ones((16,), jnp.float32)
    sub = BLK // 16
    nmax = CAP - 1040

    def st(i, lohi):
        lo, hi = lohi
        j = i // sub
        k = lax.rem(i, sub)
        sl = pl.ds(k * 16, 16)
        dv = dst_v[j, sl]
        sv = src_v[j, sl]
        plsc.addupdate_scatter(hist, [dv], ones)
        mlo = dv < HALF
        mhi = jnp.logical_not(mlo)
        plsc.store_compressed(ls.at[pl.ds(lo, 16)], sv, mask=mlo)
        plsc.store_compressed(ld.at[pl.ds(lo, 16)], dv, mask=mlo)
        plsc.store_compressed(hs.at[pl.ds(hi, 16)], sv, mask=mhi)
        plsc.store_compressed(hd.at[pl.ds(hi, 16)], dv - HALF, mask=mhi)
        nlo = jnp.sum(mlo.astype(jnp.int32))
        lo = jnp.minimum(lo + nlo, nmax)
        hi = jnp.minimum(hi + (16 - nlo), nmax)
        return lo, hi

    lo, hi = lax.fori_loop(0, STEPS * sub, st, (0, 0))

    padsrc = jnp.full((16,), N, jnp.int32)
    paddst = jnp.zeros((16,), jnp.int32)
    for k in range(64):
        ls[pl.ds(lo + k * 16, 16)] = padsrc
        ld[pl.ds(lo + k * 16, 16)] = paddst
        hs[pl.ds(hi + k * 16, 16)] = padsrc
        hd[pl.ds(hi + k * 16, 16)] = paddst
    nblo = ((lo + BLK - 1) // BLK + 7) & ~7
    nbhi = ((hi + BLK - 1) // BLK + 7) & ~7

    pltpu.sync_copy(hist, out_hbm.at[wid])
    pltpu.sync_copy(ls, srcp_hbm.at[0, wid])
    pltpu.sync_copy(ld, dstp_hbm.at[0, wid])
    pltpu.sync_copy(hs, srcp_hbm.at[1, wid])
    pltpu.sync_copy(hd, dstp_hbm.at[1, wid])
    cbuf[...] = jnp.broadcast_to(nblo, (16,))
    pltpu.sync_copy(cbuf, nblk_hbm.at[0, wid])
    cbuf[...] = jnp.broadcast_to(nbhi, (16,))
    pltpu.sync_copy(cbuf, nblk_hbm.at[1, wid])


_deg_call = functools.partial(
    pl.kernel,
    _deg_body,
    out_type=(jax.ShapeDtypeStruct((NW, NP), jnp.float32),
              jax.ShapeDtypeStruct((2, NW, CAP), jnp.int32),
              jax.ShapeDtypeStruct((2, NW, CAP), jnp.int32),
              jax.ShapeDtypeStruct((2, NW, 16), jnp.int32)),
    mesh=_sc_mesh,
    scratch_types=[
        pltpu.VMEM((STEPS, BLK), jnp.int32),
        pltpu.VMEM((STEPS, BLK), jnp.int32),
        pltpu.VMEM((NP,), jnp.float32),
        pltpu.VMEM((CAP,), jnp.int32),
        pltpu.VMEM((CAP,), jnp.int32),
        pltpu.VMEM((CAP,), jnp.int32),
        pltpu.VMEM((CAP,), jnp.int32),
        pltpu.VMEM((16,), jnp.int32),
    ],
    compiler_params=_sc_params,
)()


def _edge_body(y_hbm, srcp_hbm, dstp_hbm, nblk_hbm, out_hbm,
               srca, dsta, srcb, dstb, rows_v, zbuf, nbuf,
               y_sh, acc, sga, sgb, ssa, ssb):
    cid = lax.axis_index("c")
    sid = lax.axis_index("s")
    arpt = HALF // NSUB
    yrpt = YROWS // NSUB

    pltpu.sync_copy(y_hbm.at[pl.ds(sid * yrpt, yrpt), :],
                    y_sh.at[pl.ds(sid * yrpt, yrpt), :])

    zv = jnp.zeros((16,), jnp.float32)
    csub = C // 16

    def zz(i, _):
        r = i // csub
        cc = lax.rem(i, csub)
        zbuf[r, pl.ds(cc * 16, 16)] = zv
        return 0

    lax.fori_loop(0, 16 * csub, zz, 0)

    def zrow(i, _):
        pltpu.sync_copy(zbuf, acc.at[pl.ds(sid * arpt + i * 16, 16), :])
        return 0

    lax.fori_loop(0, arpt // 16, zrow, 0)

    pltpu.sync_copy(srcp_hbm.at[cid, 2 * sid], srca)
    pltpu.sync_copy(dstp_hbm.at[cid, 2 * sid], dsta)
    pltpu.sync_copy(srcp_hbm.at[cid, 2 * sid + 1], srcb)
    pltpu.sync_copy(dstp_hbm.at[cid, 2 * sid + 1], dstb)
    pltpu.sync_copy(nblk_hbm.at[cid, 2 * sid], nbuf)
    na = nbuf[pl.ds(0, 16)][0]
    pltpu.sync_copy(nblk_hbm.at[cid, 2 * sid + 1], nbuf)
    nb = nbuf[pl.ds(0, 16)][0]
    plsc.subcore_barrier()

    def rslice(half, k):
        return rows_v.at[pl.ds((half * GRP + k) * BLK, BLK), :]

    def run_list(src_l, dst_l, nblk):
        ngrp = nblk // GRP

        def fire_gather(half, g, sm):
            for k in range(GRP):
                pltpu.async_copy(y_sh.at[src_l.at[g * GRP + k]],
                                 rslice(half, k), sm)

        def drain_gather(half, sm):
            for k in range(GRP):
                pltpu.make_async_copy(y_hbm.at[src_l.at[0]],
                                      rslice(half, k), sm).wait()

        def fire_scatter(half, g, sm):
            for k in range(GRP):
                pltpu.async_copy(rslice(half, k),
                                 acc.at[dst_l.at[g * GRP + k]], sm, add=True)

        def drain_scatter(half, sm):
            for k in range(GRP):
                pltpu.make_async_copy(rslice(half, k),
                                      acc.at[dst_l.at[0]], sm).wait()

        @pl.when(ngrp > 0)
        def _():
            fire_gather(0, 0, sga)

            def pair(i, _):
                fire_gather(1, 2 * i + 1, sgb)
                drain_gather(0, sga)
                fire_scatter(0, 2 * i, ssa)
                drain_gather(1, sgb)
                fire_scatter(1, 2 * i + 1, ssb)
                drain_scatter(0, ssa)

                @pl.when(2 * i + 2 < ngrp)
                def _():
                    fire_gather(0, 2 * i + 2, sga)

                drain_scatter(1, ssb)
                return 0

            lax.fori_loop(0, ngrp // 2, pair, 0)

    run_list(srca, dsta, na)
    run_list(srcb, dstb, nb)
    plsc.subcore_barrier()
    pltpu.sync_copy(acc.at[pl.ds(sid * arpt, arpt), :],
                    out_hbm.at[pl.ds(cid * HALF + sid * arpt, arpt), :])


_edge_call = functools.partial(
    pl.kernel,
    _edge_body,
    out_type=jax.ShapeDtypeStruct((NP, C), jnp.float32),
    mesh=_sc_mesh,
    scratch_types=[
        pltpu.VMEM((CAPB, BLK), jnp.int32),
        pltpu.VMEM((CAPB, BLK), jnp.int32),
        pltpu.VMEM((CAPB, BLK), jnp.int32),
        pltpu.VMEM((CAPB, BLK), jnp.int32),
        pltpu.VMEM((2 * GRP * BLK, C), jnp.float32),
        pltpu.VMEM((16, C), jnp.float32),
        pltpu.VMEM((16,), jnp.int32),
        pltpu.VMEM_SHARED((YROWS, C), jnp.float32),
        pltpu.VMEM_SHARED((HALF, C), jnp.float32),
        pltpu.SemaphoreType.DMA,
        pltpu.SemaphoreType.DMA,
        pltpu.SemaphoreType.DMA,
        pltpu.SemaphoreType.DMA,
    ],
    compiler_params=_sc_params,
)()


def _gather_body(y_hbm, idx_hbm, iv_hbm, yg_hbm, ivg_hbm, idx_v, rows_v,
                 iv_v, ob_v):
    cid = lax.axis_index("c")
    sid = lax.axis_index("s")
    wid = cid * NSUB + sid

    pltpu.sync_copy(idx_hbm.at[wid], idx_v)
    pltpu.sync_copy(iv_hbm, iv_v)
    for t in range(IPW // IBLK):
        pltpu.sync_copy(y_hbm.at[idx_v.at[t]], rows_v)
        pltpu.sync_copy(rows_v,
                        yg_hbm.at[pl.ds(wid * IPW + t * IBLK, IBLK), :])
        for k in range(IBLK // 16):
            v = idx_v[t, pl.ds(k * 16, 16)]
            g = plsc.load_gather(iv_v, [v])
            ob_v[pl.ds((t * (IBLK // 16) + k) * 16, 16)] = g
    pltpu.sync_copy(ob_v, ivg_hbm.at[pl.ds(wid * IPW, IPW)])


_gather_call = functools.partial(
    pl.kernel,
    _gather_body,
    out_type=(jax.ShapeDtypeStruct((NIDXP, C), jnp.float32),
              jax.ShapeDtypeStruct((NIDXP,), jnp.float32)),
    mesh=_sc_mesh,
    scratch_types=[
        pltpu.VMEM((IPW // IBLK, IBLK), jnp.int32),
        pltpu.VMEM((IBLK, C), jnp.float32),
        pltpu.VMEM((NP,), jnp.float32),
        pltpu.VMEM((IPW,), jnp.float32),
    ],
    compiler_params=_sc_params,
)()


_MB = 512


def _mlp_body(x_ref, w1_ref, w2_ref, w3_ref, o_ref):
    h = jnp.maximum(
        jnp.dot(x_ref[...], w1_ref[...], preferred_element_type=jnp.float32),
        0.0)
    h = jnp.maximum(
        jnp.dot(h, w2_ref[...], preferred_element_type=jnp.float32), 0.0)
    o_ref[...] = jnp.dot(h, w3_ref[...], preferred_element_type=jnp.float32)


def _mlp_call(x, w1, w2, w3):
    f = x.shape[1]
    h1, h2 = w2.shape[0], w3.shape[0]
    return pl.pallas_call(
        _mlp_body,
        grid=(NP // _MB,),
        in_specs=[
            pl.BlockSpec((_MB, f), lambda i: (i, 0)),
            pl.BlockSpec((f, h1), lambda i: (0, 0)),
            pl.BlockSpec((h1, h2), lambda i: (0, 0)),
            pl.BlockSpec((h2, C), lambda i: (0, 0)),
        ],
        out_specs=pl.BlockSpec((_MB, C), lambda i: (i, 0)),
        out_shape=jax.ShapeDtypeStruct((NP, C), jnp.float32),
    )(x, w1, w2, w3)


_RB = 512


def _prep_body(hist_ref, l_ref, a1_ref, bv_ref, y0_ref, iv_ref):
    i = pl.program_id(0)
    cnt = jnp.sum(hist_ref[...], axis=1, keepdims=True)
    deg = cnt + 1.0
    dis = lax.rsqrt(deg)
    row = i * _RB + lax.broadcasted_iota(jnp.int32, (_RB, 1), 0)
    mask = (row < N).astype(jnp.float32)
    lv = l_ref[...]
    a1_ref[...] = (1.0 - ALPHA) * dis * dis * mask
    bv_ref[...] = ALPHA * dis * lv
    y0_ref[...] = dis * lv
    iv_ref[...] = jnp.sqrt(deg)


def _prep_call(hist_t, l):
    return pl.pallas_call(
        _prep_body,
        grid=(NP // _RB,),
        in_specs=[
            pl.BlockSpec((_RB, NW), lambda i: (i, 0)),
            pl.BlockSpec((_RB, C), lambda i: (i, 0)),
        ],
        out_specs=[
            pl.BlockSpec((_RB, 1), lambda i: (i, 0)),
            pl.BlockSpec((_RB, C), lambda i: (i, 0)),
            pl.BlockSpec((_RB, C), lambda i: (i, 0)),
            pl.BlockSpec((_RB, 1), lambda i: (i, 0)),
        ],
        out_shape=[
            jax.ShapeDtypeStruct((NP, 1), jnp.float32),
            jax.ShapeDtypeStruct((NP, C), jnp.float32),
            jax.ShapeDtypeStruct((NP, C), jnp.float32),
            jax.ShapeDtypeStruct((NP, 1), jnp.float32),
        ],
    )(hist_t, l)


_CB = 1024


def _comb_body(p_ref, y_ref, a1_ref, bv_ref, o_ref):
    s = p_ref[...] + y_ref[...]
    o_ref[...] = a1_ref[...] * s + bv_ref[...]


def _comb_call(p, y, a1, bv):
    return pl.pallas_call(
        _comb_body,
        grid=(NP // _CB,),
        in_specs=[
            pl.BlockSpec((_CB, C), lambda i: (i, 0)),
            pl.BlockSpec((_CB, C), lambda i: (i, 0)),
            pl.BlockSpec((_CB, 1), lambda i: (i, 0)),
            pl.BlockSpec((_CB, C), lambda i: (i, 0)),
        ],
        out_specs=pl.BlockSpec((_CB, C), lambda i: (i, 0)),
        out_shape=jax.ShapeDtypeStruct((NP, C), jnp.float32),
    )(p, y, a1, bv)


_SB = 512


def _lsm_body(y_ref, iv_ref, o_ref):
    z = y_ref[...] * iv_ref[...]
    m = jnp.max(z, axis=1, keepdims=True)
    e = jnp.exp(z - m)
    s = jnp.sum(e, axis=1, keepdims=True)
    o_ref[...] = (z - m) - jnp.log(s)


def _lsm_call(yg, ivg):
    return pl.pallas_call(
        _lsm_body,
        grid=(NIDXP // _SB,),
        in_specs=[
            pl.BlockSpec((_SB, C), lambda i: (i, 0)),
            pl.BlockSpec((_SB, 1), lambda i: (i, 0)),
        ],
        out_specs=pl.BlockSpec((_SB, C), lambda i: (i, 0)),
        out_shape=jax.ShapeDtypeStruct((NIDXP, C), jnp.float32),
    )(yg, ivg)


def kernel(attr_matrix, idx, W1, W2, W3, edge_index):
    attr_p = jnp.pad(attr_matrix, ((0, NP - N), (0, 0)))
    src = jnp.concatenate(
        [edge_index[0], jnp.zeros((EP - E,), jnp.int32)]).reshape(
            NW, STEPS, BLK)
    dst = jnp.concatenate(
        [edge_index[1], jnp.full((EP - E,), N, jnp.int32)]).reshape(
            NW, STEPS, BLK)
    idx_p = jnp.concatenate(
        [idx, jnp.zeros((NIDXP - NIDX,), jnp.int32)]).reshape(
            NW, IPW // IBLK, IBLK)

    l = _mlp_call(attr_p, W1, W2, W3)
    hist, srcp, dstp, nblk = _deg_call(src, dst)
    srcp = srcp.reshape(2, NW, CAPB, BLK)
    dstp = dstp.reshape(2, NW, CAPB, BLK)
    a1, bv, y, iv = _prep_call(hist.T, l)
    for _ in range(NITER):
        p = _edge_call(y, srcp, dstp, nblk)
        y = _comb_call(p, y, a1, bv)
    yg, ivg = _gather_call(y, idx_p, iv.reshape(NP))
    out = _lsm_call(yg, ivg.reshape(NIDXP, 1))
    return out[:NIDX]


# speedup vs baseline: 1.1118x; 1.0383x over previous
import functools

import jax
import jax.numpy as jnp
from jax import lax
from jax.experimental import pallas as pl
from jax.experimental.pallas import tpu as pltpu
from jax.experimental.pallas import tpu_sc as plsc

N = 10000
C = 64
E = 320000
NIDX = 5000
ALPHA = 0.1
NITER = 10

NCORES = 2
NSUB = 16
NW = NCORES * NSUB
NP = 10240
RPT = NP // NSUB
BLK = 128
GRP = 2
YROWS = 10016
STEPS = 80
EP = NW * STEPS * BLK

NIDXP = 5120
IPW = NIDXP // NW
IBLK = 80

HALF = NP // 2
CAPB = 52
CAP = CAPB * BLK

_sc_mesh = plsc.VectorSubcoreMesh(core_axis_name="c", subcore_axis_name="s")
_sc_params = pltpu.CompilerParams(needs_layout_passes=False,
                                  use_tc_tiling_on_sc=False)


def _deg_body(src_hbm, dst_hbm, out_hbm, srcp_hbm, dstp_hbm, nblk_hbm,
              src_v, dst_v, hist, ls, ld, hs, hd, cbuf):
    cid = lax.axis_index("c")
    sid = lax.axis_index("s")
    wid = cid * NSUB + sid

    zv = jnp.zeros((16,), jnp.float32)

    def zh(i, _):
        hist[pl.ds(i * 16, 16)] = zv
        return 0

    lax.fori_loop(0, NP // 16, zh, 0)
    pltpu.sync_copy(src_hbm.at[wid], src_v)
    pltpu.sync_copy(dst_hbm.at[wid], dst_v)

    ones = jnp.ones((16,), jnp.float32)
    sub = BLK // 16
    nmax = CAP - 1040

    def st(i, lohi):
        lo, hi = lohi
        j = i // sub
        k = lax.rem(i, sub)
        sl = pl.ds(k * 16, 16)
        dv = dst_v[j, sl]
        sv = src_v[j, sl]
        plsc.addupdate_scatter(hist, [dv], ones)
        mlo = dv < HALF
        mhi = jnp.logical_not(mlo)
        plsc.store_compressed(ls.at[pl.ds(lo, 16)], sv, mask=mlo)
        plsc.store_compressed(ld.at[pl.ds(lo, 16)], dv, mask=mlo)
        plsc.store_compressed(hs.at[pl.ds(hi, 16)], sv, mask=mhi)
        plsc.store_compressed(hd.at[pl.ds(hi, 16)], dv - HALF, mask=mhi)
        nlo = jnp.sum(mlo.astype(jnp.int32))
        lo = jnp.minimum(lo + nlo, nmax)
        hi = jnp.minimum(hi + (16 - nlo), nmax)
        return lo, hi

    lo, hi = lax.fori_loop(0, STEPS * sub, st, (0, 0))

    padsrc = jnp.full((16,), N, jnp.int32)
    paddst = jnp.zeros((16,), jnp.int32)
    for k in range(64):
        ls[pl.ds(lo + k * 16, 16)] = padsrc
        ld[pl.ds(lo + k * 16, 16)] = paddst
        hs[pl.ds(hi + k * 16, 16)] = padsrc
        hd[pl.ds(hi + k * 16, 16)] = paddst
    nblo = ((lo + BLK - 1) // BLK + 7) & ~7
    nbhi = ((hi + BLK - 1) // BLK + 7) & ~7

    pltpu.sync_copy(hist, out_hbm.at[wid])
    pltpu.sync_copy(ls, srcp_hbm.at[0, wid])
    pltpu.sync_copy(ld, dstp_hbm.at[0, wid])
    pltpu.sync_copy(hs, srcp_hbm.at[1, wid])
    pltpu.sync_copy(hd, dstp_hbm.at[1, wid])
    cbuf[...] = jnp.broadcast_to(nblo, (16,))
    pltpu.sync_copy(cbuf, nblk_hbm.at[0, wid])
    cbuf[...] = jnp.broadcast_to(nbhi, (16,))
    pltpu.sync_copy(cbuf, nblk_hbm.at[1, wid])


_deg_call = functools.partial(
    pl.kernel,
    _deg_body,
    out_type=(jax.ShapeDtypeStruct((NW, NP), jnp.float32),
              jax.ShapeDtypeStruct((2, NW, CAP), jnp.int32),
              jax.ShapeDtypeStruct((2, NW, CAP), jnp.int32),
              jax.ShapeDtypeStruct((2, NW, 16), jnp.int32)),
    mesh=_sc_mesh,
    scratch_types=[
        pltpu.VMEM((STEPS, BLK), jnp.int32),
        pltpu.VMEM((STEPS, BLK), jnp.int32),
        pltpu.VMEM((NP,), jnp.float32),
        pltpu.VMEM((CAP,), jnp.int32),
        pltpu.VMEM((CAP,), jnp.int32),
        pltpu.VMEM((CAP,), jnp.int32),
        pltpu.VMEM((CAP,), jnp.int32),
        pltpu.VMEM((16,), jnp.int32),
    ],
    compiler_params=_sc_params,
)()


def _edge_body(y_hbm, srcp_hbm, dstp_hbm, nblk_hbm, out_hbm,
               srca, dsta, srcb, dstb, rows_v, zbuf, nbuf,
               y_sh, acc, sga, sgb, ssa, ssb):
    cid = lax.axis_index("c")
    sid = lax.axis_index("s")
    arpt = HALF // NSUB
    yrpt = YROWS // NSUB

    cpy = pltpu.async_copy(y_hbm.at[pl.ds(sid * yrpt, yrpt), :],
                           y_sh.at[pl.ds(sid * yrpt, yrpt), :], sga)
    cps = [pltpu.async_copy(srcp_hbm.at[cid, 2 * sid], srca, sgb),
           pltpu.async_copy(dstp_hbm.at[cid, 2 * sid], dsta, sgb),
           pltpu.async_copy(srcp_hbm.at[cid, 2 * sid + 1], srcb, sgb),
           pltpu.async_copy(dstp_hbm.at[cid, 2 * sid + 1], dstb, sgb)]

    zv = jnp.zeros((16,), jnp.float32)
    csub = C // 16

    def zz(i, _):
        r = i // csub
        cc = lax.rem(i, csub)
        zbuf[r, pl.ds(cc * 16, 16)] = zv
        return 0

    lax.fori_loop(0, 16 * csub, zz, 0)

    def zrow(i, _):
        pltpu.sync_copy(zbuf, acc.at[pl.ds(sid * arpt + i * 16, 16), :])
        return 0

    lax.fori_loop(0, arpt // 16, zrow, 0)

    pltpu.sync_copy(nblk_hbm.at[cid, 2 * sid], nbuf)
    na = nbuf[pl.ds(0, 16)][0]
    pltpu.sync_copy(nblk_hbm.at[cid, 2 * sid + 1], nbuf)
    nb = nbuf[pl.ds(0, 16)][0]
    cpy.wait()
    for cp in cps:
        cp.wait()
    plsc.subcore_barrier()

    def rslice(half, k):
        return rows_v.at[pl.ds((half * GRP + k) * BLK, BLK), :]

    def run_list(src_l, dst_l, nblk):
        ngrp = nblk // GRP

        def fire_gather(half, g, sm):
            for k in range(GRP):
                pltpu.async_copy(y_sh.at[src_l.at[g * GRP + k]],
                                 rslice(half, k), sm)

        def drain_gather(half, sm):
            for k in range(GRP):
                pltpu.make_async_copy(y_hbm.at[src_l.at[0]],
                                      rslice(half, k), sm).wait()

        def fire_scatter(half, g, sm):
            for k in range(GRP):
                pltpu.async_copy(rslice(half, k),
                                 acc.at[dst_l.at[g * GRP + k]], sm, add=True)

        def drain_scatter(half, sm):
            for k in range(GRP):
                pltpu.make_async_copy(rslice(half, k),
                                      acc.at[dst_l.at[0]], sm).wait()

        @pl.when(ngrp > 0)
        def _():
            fire_gather(0, 0, sga)

            def pair(i, _):
                fire_gather(1, 2 * i + 1, sgb)
                drain_gather(0, sga)
                fire_scatter(0, 2 * i, ssa)
                drain_gather(1, sgb)
                fire_scatter(1, 2 * i + 1, ssb)
                drain_scatter(0, ssa)

                @pl.when(2 * i + 2 < ngrp)
                def _():
                    fire_gather(0, 2 * i + 2, sga)

                drain_scatter(1, ssb)
                return 0

            lax.fori_loop(0, ngrp // 2, pair, 0)

    run_list(srca, dsta, na)
    run_list(srcb, dstb, nb)
    plsc.subcore_barrier()
    pltpu.sync_copy(acc.at[pl.ds(sid * arpt, arpt), :],
                    out_hbm.at[pl.ds(cid * HALF + sid * arpt, arpt), :])


_edge_call = functools.partial(
    pl.kernel,
    _edge_body,
    out_type=jax.ShapeDtypeStruct((NP, C), jnp.float32),
    mesh=_sc_mesh,
    scratch_types=[
        pltpu.VMEM((CAPB, BLK), jnp.int32),
        pltpu.VMEM((CAPB, BLK), jnp.int32),
        pltpu.VMEM((CAPB, BLK), jnp.int32),
        pltpu.VMEM((CAPB, BLK), jnp.int32),
        pltpu.VMEM((2 * GRP * BLK, C), jnp.float32),
        pltpu.VMEM((16, C), jnp.float32),
        pltpu.VMEM((16,), jnp.int32),
        pltpu.VMEM_SHARED((YROWS, C), jnp.float32),
        pltpu.VMEM_SHARED((HALF, C), jnp.float32),
        pltpu.SemaphoreType.DMA,
        pltpu.SemaphoreType.DMA,
        pltpu.SemaphoreType.DMA,
        pltpu.SemaphoreType.DMA,
    ],
    compiler_params=_sc_params,
)()


def _gather_body(y_hbm, idx_hbm, iv_hbm, yg_hbm, ivg_hbm, idx_v, rows_v,
                 iv_v, ob_v):
    cid = lax.axis_index("c")
    sid = lax.axis_index("s")
    wid = cid * NSUB + sid

    pltpu.sync_copy(idx_hbm.at[wid], idx_v)
    pltpu.sync_copy(iv_hbm, iv_v)
    for t in range(IPW // IBLK):
        pltpu.sync_copy(y_hbm.at[idx_v.at[t]], rows_v)
        pltpu.sync_copy(rows_v,
                        yg_hbm.at[pl.ds(wid * IPW + t * IBLK, IBLK), :])
        for k in range(IBLK // 16):
            v = idx_v[t, pl.ds(k * 16, 16)]
            g = plsc.load_gather(iv_v, [v])
            ob_v[pl.ds((t * (IBLK // 16) + k) * 16, 16)] = g
    pltpu.sync_copy(ob_v, ivg_hbm.at[pl.ds(wid * IPW, IPW)])


_gather_call = functools.partial(
    pl.kernel,
    _gather_body,
    out_type=(jax.ShapeDtypeStruct((NIDXP, C), jnp.float32),
              jax.ShapeDtypeStruct((NIDXP,), jnp.float32)),
    mesh=_sc_mesh,
    scratch_types=[
        pltpu.VMEM((IPW // IBLK, IBLK), jnp.int32),
        pltpu.VMEM((IBLK, C), jnp.float32),
        pltpu.VMEM((NP,), jnp.float32),
        pltpu.VMEM((IPW,), jnp.float32),
    ],
    compiler_params=_sc_params,
)()


_MB = 512


def _mlp_body(x_ref, w1_ref, w2_ref, w3_ref, o_ref):
    h = jnp.maximum(
        jnp.dot(x_ref[...], w1_ref[...], preferred_element_type=jnp.float32),
        0.0)
    h = jnp.maximum(
        jnp.dot(h, w2_ref[...], preferred_element_type=jnp.float32), 0.0)
    o_ref[...] = jnp.dot(h, w3_ref[...], preferred_element_type=jnp.float32)


def _mlp_call(x, w1, w2, w3):
    f = x.shape[1]
    h1, h2 = w2.shape[0], w3.shape[0]
    return pl.pallas_call(
        _mlp_body,
        grid=(NP // _MB,),
        in_specs=[
            pl.BlockSpec((_MB, f), lambda i: (i, 0)),
            pl.BlockSpec((f, h1), lambda i: (0, 0)),
            pl.BlockSpec((h1, h2), lambda i: (0, 0)),
            pl.BlockSpec((h2, C), lambda i: (0, 0)),
        ],
        out_specs=pl.BlockSpec((_MB, C), lambda i: (i, 0)),
        out_shape=jax.ShapeDtypeStruct((NP, C), jnp.float32),
    )(x, w1, w2, w3)


_RB = 512


def _prep_body(hist_ref, l_ref, a1_ref, bv_ref, y0_ref, iv_ref):
    i = pl.program_id(0)
    cnt = jnp.sum(hist_ref[...], axis=1, keepdims=True)
    deg = cnt + 1.0
    dis = lax.rsqrt(deg)
    row = i * _RB + lax.broadcasted_iota(jnp.int32, (_RB, 1), 0)
    mask = (row < N).astype(jnp.float32)
    lv = l_ref[...]
    a1_ref[...] = (1.0 - ALPHA) * dis * dis * mask
    bv_ref[...] = ALPHA * dis * lv
    y0_ref[...] = dis * lv
    iv_ref[...] = jnp.sqrt(deg)


def _prep_call(hist_t, l):
    return pl.pallas_call(
        _prep_body,
        grid=(NP // _RB,),
        in_specs=[
            pl.BlockSpec((_RB, NW), lambda i: (i, 0)),
            pl.BlockSpec((_RB, C), lambda i: (i, 0)),
        ],
        out_specs=[
            pl.BlockSpec((_RB, 1), lambda i: (i, 0)),
            pl.BlockSpec((_RB, C), lambda i: (i, 0)),
            pl.BlockSpec((_RB, C), lambda i: (i, 0)),
            pl.BlockSpec((_RB, 1), lambda i: (i, 0)),
        ],
        out_shape=[
            jax.ShapeDtypeStruct((NP, 1), jnp.float32),
            jax.ShapeDtypeStruct((NP, C), jnp.float32),
            jax.ShapeDtypeStruct((NP, C), jnp.float32),
            jax.ShapeDtypeStruct((NP, 1), jnp.float32),
        ],
    )(hist_t, l)


_CB = 1024


def _comb_body(p_ref, y_ref, a1_ref, bv_ref, o_ref):
    s = p_ref[...] + y_ref[...]
    o_ref[...] = a1_ref[...] * s + bv_ref[...]


def _comb_call(p, y, a1, bv):
    return pl.pallas_call(
        _comb_body,
        grid=(NP // _CB,),
        in_specs=[
            pl.BlockSpec((_CB, C), lambda i: (i, 0)),
            pl.BlockSpec((_CB, C), lambda i: (i, 0)),
            pl.BlockSpec((_CB, 1), lambda i: (i, 0)),
            pl.BlockSpec((_CB, C), lambda i: (i, 0)),
        ],
        out_specs=pl.BlockSpec((_CB, C), lambda i: (i, 0)),
        out_shape=jax.ShapeDtypeStruct((NP, C), jnp.float32),
    )(p, y, a1, bv)


_SB = 512


def _lsm_body(y_ref, iv_ref, o_ref):
    z = y_ref[...] * iv_ref[...]
    m = jnp.max(z, axis=1, keepdims=True)
    e = jnp.exp(z - m)
    s = jnp.sum(e, axis=1, keepdims=True)
    o_ref[...] = (z - m) - jnp.log(s)


def _lsm_call(yg, ivg):
    return pl.pallas_call(
        _lsm_body,
        grid=(NIDXP // _SB,),
        in_specs=[
            pl.BlockSpec((_SB, C), lambda i: (i, 0)),
            pl.BlockSpec((_SB, 1), lambda i: (i, 0)),
        ],
        out_specs=pl.BlockSpec((_SB, C), lambda i: (i, 0)),
        out_shape=jax.ShapeDtypeStruct((NIDXP, C), jnp.float32),
    )(yg, ivg)


def kernel(attr_matrix, idx, W1, W2, W3, edge_index):
    attr_p = jnp.pad(attr_matrix, ((0, NP - N), (0, 0)))
    src = jnp.concatenate(
        [edge_index[0], jnp.zeros((EP - E,), jnp.int32)]).reshape(
            NW, STEPS, BLK)
    dst = jnp.concatenate(
        [edge_index[1], jnp.full((EP - E,), N, jnp.int32)]).reshape(
            NW, STEPS, BLK)
    idx_p = jnp.concatenate(
        [idx, jnp.zeros((NIDXP - NIDX,), jnp.int32)]).reshape(
            NW, IPW // IBLK, IBLK)

    l = _mlp_call(attr_p, W1, W2, W3)
    hist, srcp, dstp, nblk = _deg_call(src, dst)
    srcp = srcp.reshape(2, NW, CAPB, BLK)
    dstp = dstp.reshape(2, NW, CAPB, BLK)
    a1, bv, y, iv = _prep_call(hist.T, l)
    for _ in range(NITER):
        p = _edge_call(y, srcp, dstp, nblk)
        y = _comb_call(p, y, a1, bv)
    yg, ivg = _gather_call(y, idx_p, iv.reshape(NP))
    out = _lsm_call(yg, ivg.reshape(NIDXP, 1))
    return out[:NIDX]
